# baseline (speedup 1.0000x reference)
import functools

import jax
import jax.numpy as jnp
import numpy as np
from jax import lax
from jax.experimental import pallas as pl
from jax.experimental.pallas import tpu as pltpu
from jax.experimental.pallas import tpu_sc as plsc

_RATIO = 0.75


def _rotl(x, d):
    return ((x << np.uint32(d)) | (x >> np.uint32(32 - d))).astype(np.uint32)


def _threefry2x32(k0, k1, x0, x1):
    x0 = np.asarray(x0, np.uint32).copy()
    x1 = np.asarray(x1, np.uint32).copy()
    ks = [
        np.uint32(k0),
        np.uint32(k1),
        np.uint32(np.uint32(0x1BD11BDA) ^ np.uint32(k0) ^ np.uint32(k1)),
    ]
    rot = [[13, 15, 26, 6], [17, 29, 16, 24]]
    x0 = (x0 + ks[0]).astype(np.uint32)
    x1 = (x1 + ks[1]).astype(np.uint32)
    for i in range(5):
        for r in rot[i % 2]:
            x0 = (x0 + x1).astype(np.uint32)
            x1 = _rotl(x1, r)
            x1 = x1 ^ x0
        x0 = (x0 + ks[(i + 1) % 3]).astype(np.uint32)
        x1 = (x1 + ks[(i + 2) % 3] + np.uint32(i + 1)).astype(np.uint32)
    return x0, x1


def _fixed_noise(B, N):
    fk0, fk1 = _threefry2x32(0, 0, np.zeros(1, np.uint32), np.ones(1, np.uint32))
    n = B * N
    x0, x1 = _threefry2x32(
        fk0[0], fk1[0], np.zeros(n, np.uint32), np.arange(n, dtype=np.uint32)
    )
    bits = x0 ^ x1
    f = ((bits >> np.uint32(9)) | np.uint32(0x3F800000)).view(np.float32)
    return np.maximum(np.float32(0.0), f - np.float32(1.0)).reshape(B, N)


@functools.lru_cache(maxsize=None)
def _shuffle_constants(B, N):
    noise = _fixed_noise(B, N)
    n_keep = int(N * (1 - _RATIO))
    ids_shuffle = np.argsort(noise, axis=1, kind="stable").astype(np.int32)
    ids_restore = np.argsort(ids_shuffle, axis=1, kind="stable").astype(np.int32)
    ids_keep = ids_shuffle[:, :n_keep]
    mask = np.ones((B, N), dtype=np.float32)
    mask[:, :n_keep] = 0.0
    mask = np.take_along_axis(mask, ids_restore, axis=1)
    flat_ids = (np.arange(B, dtype=np.int32)[:, None] * N + ids_keep).reshape(-1)
    return ids_keep, ids_restore, mask, flat_ids


_NBUF = 3


@functools.lru_cache(maxsize=None)
def _make_shuffle(B, N, D, n_keep, ch):
    R = B * n_keep
    info = plsc.get_sparse_core_info()
    NC, NS = info.num_cores, info.num_subcores
    NW = NC * NS
    rpw = R // NW
    assert R % NW == 0 and rpw % ch == 0 and ch % 8 == 0 and ch <= 128
    assert B % NW == 0
    bpw = B // NW
    nch = rpw // ch
    nbuf = min(_NBUF, nch)
    mesh = plsc.VectorSubcoreMesh(core_axis_name="c", subcore_axis_name="s")

    @functools.partial(
        pl.kernel,
        out_type=[
            jax.ShapeDtypeStruct((R, D), jnp.float32),
            jax.ShapeDtypeStruct((B, N), jnp.float32),
            jax.ShapeDtypeStruct((B, n_keep), jnp.int32),
            jax.ShapeDtypeStruct((B, N), jnp.int32),
        ],
        mesh=mesh,
        scratch_types=[
            pltpu.VMEM((nch, ch), jnp.int32),
            pltpu.VMEM((bpw, N), jnp.float32),
            pltpu.VMEM((bpw, n_keep), jnp.int32),
            pltpu.VMEM((bpw, N), jnp.int32),
        ]
        + [pltpu.VMEM((ch, D), jnp.float32)] * nbuf
        + [pltpu.SemaphoreType.DMA] * (2 * nbuf),
    )
    def shuffle(
        table_hbm,
        idx_hbm,
        maskc_hbm,
        keepc_hbm,
        restc_hbm,
        out_hbm,
        mask_hbm,
        keep_hbm,
        rest_hbm,
        idx_v,
        mask_v,
        keep_v,
        rest_v,
        *bufs_sems,
    ):
        bufs = bufs_sems[:nbuf]
        gsem = bufs_sems[nbuf : 2 * nbuf]
        ssem = bufs_sems[2 * nbuf :]
        wid = lax.axis_index("s") * NC + lax.axis_index("c")
        base = wid * rpw
        pltpu.sync_copy(idx_hbm.at[wid], idx_v)
        gcp = [None] * nch
        scp = [None] * nch
        for c in range(nbuf):
            gcp[c] = pltpu.async_copy(
                table_hbm.at[idx_v.at[c]], bufs[c], gsem[c]
            )
        brow = wid * bpw
        pltpu.sync_copy(maskc_hbm.at[pl.ds(brow, bpw)], mask_v)
        pltpu.sync_copy(mask_v, mask_hbm.at[pl.ds(brow, bpw)])
        pltpu.sync_copy(keepc_hbm.at[pl.ds(brow, bpw)], keep_v)
        pltpu.sync_copy(keep_v, keep_hbm.at[pl.ds(brow, bpw)])
        pltpu.sync_copy(restc_hbm.at[pl.ds(brow, bpw)], rest_v)
        pltpu.sync_copy(rest_v, rest_hbm.at[pl.ds(brow, bpw)])
        for c in range(nch):
            gcp[c].wait()
            scp[c] = pltpu.async_copy(
                bufs[c % nbuf], out_hbm.at[pl.ds(base + c * ch, ch)], ssem[c % nbuf]
            )
            if c + nbuf < nch:
                scp[c].wait()
                gcp[c + nbuf] = pltpu.async_copy(
                    table_hbm.at[idx_v.at[c + nbuf]], bufs[c % nbuf], gsem[c % nbuf]
                )
        for c in range(max(0, nch - nbuf), nch):
            scp[c].wait()

    return shuffle


def kernel(patches):
    B, N, D = patches.shape
    n_keep = int(N * (1 - _RATIO))
    ids_keep, ids_restore, mask, flat_ids = _shuffle_constants(B, N)
    R = B * n_keep
    info = plsc.get_sparse_core_info()
    NW = info.num_cores * info.num_subcores
    rpw = R // NW
    ch = max(
        c
        for c in range(8, min(rpw, 128) + 1, 8)
        if rpw % c == 0 and (_NBUF * c * D + rpw) * 4 <= 500_000
    )
    idx = jnp.asarray(flat_ids.reshape(NW, rpw // ch, ch))
    table = patches.reshape(B * N, D)
    out, mask_o, keep_o, rest_o = _make_shuffle(B, N, D, n_keep, ch)(
        table,
        idx,
        jnp.asarray(mask),
        jnp.asarray(ids_keep),
        jnp.asarray(ids_restore),
    )
    patches_keep = out.reshape(B, n_keep, D)
    return (patches_keep, mask_o, keep_o, rest_o)


# speedup vs baseline: 1.0128x; 1.0128x over previous
import functools

import jax
import jax.numpy as jnp
import numpy as np
from jax import lax
from jax.experimental import pallas as pl
from jax.experimental.pallas import tpu as pltpu
from jax.experimental.pallas import tpu_sc as plsc

_RATIO = 0.75


def _rotl(x, d):
    return ((x << np.uint32(d)) | (x >> np.uint32(32 - d))).astype(np.uint32)


def _threefry2x32(k0, k1, x0, x1):
    x0 = np.asarray(x0, np.uint32).copy()
    x1 = np.asarray(x1, np.uint32).copy()
    ks = [
        np.uint32(k0),
        np.uint32(k1),
        np.uint32(np.uint32(0x1BD11BDA) ^ np.uint32(k0) ^ np.uint32(k1)),
    ]
    rot = [[13, 15, 26, 6], [17, 29, 16, 24]]
    x0 = (x0 + ks[0]).astype(np.uint32)
    x1 = (x1 + ks[1]).astype(np.uint32)
    for i in range(5):
        for r in rot[i % 2]:
            x0 = (x0 + x1).astype(np.uint32)
            x1 = _rotl(x1, r)
            x1 = x1 ^ x0
        x0 = (x0 + ks[(i + 1) % 3]).astype(np.uint32)
        x1 = (x1 + ks[(i + 2) % 3] + np.uint32(i + 1)).astype(np.uint32)
    return x0, x1


def _fixed_noise(B, N):
    fk0, fk1 = _threefry2x32(0, 0, np.zeros(1, np.uint32), np.ones(1, np.uint32))
    n = B * N
    x0, x1 = _threefry2x32(
        fk0[0], fk1[0], np.zeros(n, np.uint32), np.arange(n, dtype=np.uint32)
    )
    bits = x0 ^ x1
    f = ((bits >> np.uint32(9)) | np.uint32(0x3F800000)).view(np.float32)
    return np.maximum(np.float32(0.0), f - np.float32(1.0)).reshape(B, N)


@functools.lru_cache(maxsize=None)
def _shuffle_constants(B, N):
    noise = _fixed_noise(B, N)
    n_keep = int(N * (1 - _RATIO))
    ids_shuffle = np.argsort(noise, axis=1, kind="stable").astype(np.int32)
    ids_restore = np.argsort(ids_shuffle, axis=1, kind="stable").astype(np.int32)
    ids_keep = ids_shuffle[:, :n_keep]
    mask = np.ones((B, N), dtype=np.float32)
    mask[:, :n_keep] = 0.0
    mask = np.take_along_axis(mask, ids_restore, axis=1)
    flat_ids = (np.arange(B, dtype=np.int32)[:, None] * N + ids_keep).reshape(-1)
    return ids_keep, ids_restore, mask, flat_ids


_NBUF = 3


@functools.lru_cache(maxsize=None)
def _make_shuffle(B, N, D, n_keep, ch):
    R = B * n_keep
    info = plsc.get_sparse_core_info()
    NC, NS = info.num_cores, info.num_subcores
    NW = NC * NS
    rpw = R // NW
    assert R % NW == 0 and rpw % ch == 0 and ch % 8 == 0 and ch <= 128
    assert B % NW == 0
    bpw = B // NW
    nch = rpw // ch
    nbuf = min(_NBUF, nch)
    mesh = plsc.VectorSubcoreMesh(core_axis_name="c", subcore_axis_name="s")

    @functools.partial(
        pl.kernel,
        out_type=[
            jax.ShapeDtypeStruct((R, D), jnp.float32),
            jax.ShapeDtypeStruct((B, N), jnp.float32),
            jax.ShapeDtypeStruct((B, n_keep), jnp.int32),
            jax.ShapeDtypeStruct((B, N), jnp.int32),
        ],
        mesh=mesh,
        scratch_types=[
            pltpu.VMEM((nch, ch), jnp.int32),
            pltpu.VMEM((bpw, N), jnp.float32),
            pltpu.VMEM((bpw, n_keep), jnp.int32),
            pltpu.VMEM((bpw, N), jnp.int32),
            pltpu.SemaphoreType.DMA,
            pltpu.SemaphoreType.DMA,
        ]
        + [pltpu.VMEM((ch, D), jnp.float32)] * nbuf
        + [pltpu.SemaphoreType.DMA] * (2 * nbuf),
    )
    def shuffle(
        table_hbm,
        idx_hbm,
        maskc_hbm,
        keepc_hbm,
        restc_hbm,
        out_hbm,
        mask_hbm,
        keep_hbm,
        rest_hbm,
        idx_v,
        mask_v,
        keep_v,
        rest_v,
        isem,
        osem,
        *bufs_sems,
    ):
        bufs = bufs_sems[:nbuf]
        gsem = bufs_sems[nbuf : 2 * nbuf]
        ssem = bufs_sems[2 * nbuf :]
        wid = lax.axis_index("s") * NC + lax.axis_index("c")
        base = wid * rpw
        brow = wid * bpw
        pltpu.sync_copy(idx_hbm.at[wid], idx_v)
        gcp = [None] * nch
        scp = [None] * nch
        for c in range(nbuf):
            gcp[c] = pltpu.async_copy(
                table_hbm.at[idx_v.at[c]], bufs[c], gsem[c]
            )
        icp = [
            pltpu.async_copy(maskc_hbm.at[pl.ds(brow, bpw)], mask_v, isem),
            pltpu.async_copy(keepc_hbm.at[pl.ds(brow, bpw)], keep_v, isem),
            pltpu.async_copy(restc_hbm.at[pl.ds(brow, bpw)], rest_v, isem),
        ]
        for c in range(nch):
            gcp[c].wait()
            scp[c] = pltpu.async_copy(
                bufs[c % nbuf], out_hbm.at[pl.ds(base + c * ch, ch)], ssem[c % nbuf]
            )
            if c + nbuf < nch:
                scp[c].wait()
                gcp[c + nbuf] = pltpu.async_copy(
                    table_hbm.at[idx_v.at[c + nbuf]], bufs[c % nbuf], gsem[c % nbuf]
                )
        for cp in icp:
            cp.wait()
        ocp = [
            pltpu.async_copy(mask_v, mask_hbm.at[pl.ds(brow, bpw)], osem),
            pltpu.async_copy(keep_v, keep_hbm.at[pl.ds(brow, bpw)], osem),
            pltpu.async_copy(rest_v, rest_hbm.at[pl.ds(brow, bpw)], osem),
        ]
        for c in range(max(0, nch - nbuf), nch):
            scp[c].wait()
        for cp in ocp:
            cp.wait()

    return shuffle


def kernel(patches):
    B, N, D = patches.shape
    n_keep = int(N * (1 - _RATIO))
    ids_keep, ids_restore, mask, flat_ids = _shuffle_constants(B, N)
    R = B * n_keep
    info = plsc.get_sparse_core_info()
    NW = info.num_cores * info.num_subcores
    rpw = R // NW
    ch = max(
        c
        for c in range(8, min(rpw, 128) + 1, 8)
        if rpw % c == 0 and (_NBUF * c * D + rpw) * 4 <= 500_000
    )
    idx = jnp.asarray(flat_ids.reshape(NW, rpw // ch, ch))
    table = patches.reshape(B * N, D)
    out, mask_o, keep_o, rest_o = _make_shuffle(B, N, D, n_keep, ch)(
        table,
        idx,
        jnp.asarray(mask),
        jnp.asarray(ids_keep),
        jnp.asarray(ids_restore),
    )
    patches_keep = out.reshape(B, n_keep, D)
    return (patches_keep, mask_o, keep_o, rest_o)


# speedup vs baseline: 1.0469x; 1.0336x over previous
import functools

import jax
import jax.numpy as jnp
import numpy as np
from jax import lax
from jax.experimental import pallas as pl
from jax.experimental.pallas import tpu as pltpu
from jax.experimental.pallas import tpu_sc as plsc

_RATIO = 0.75


def _rotl(x, d):
    return ((x << np.uint32(d)) | (x >> np.uint32(32 - d))).astype(np.uint32)


def _threefry2x32(k0, k1, x0, x1):
    x0 = np.asarray(x0, np.uint32).copy()
    x1 = np.asarray(x1, np.uint32).copy()
    ks = [
        np.uint32(k0),
        np.uint32(k1),
        np.uint32(np.uint32(0x1BD11BDA) ^ np.uint32(k0) ^ np.uint32(k1)),
    ]
    rot = [[13, 15, 26, 6], [17, 29, 16, 24]]
    x0 = (x0 + ks[0]).astype(np.uint32)
    x1 = (x1 + ks[1]).astype(np.uint32)
    for i in range(5):
        for r in rot[i % 2]:
            x0 = (x0 + x1).astype(np.uint32)
            x1 = _rotl(x1, r)
            x1 = x1 ^ x0
        x0 = (x0 + ks[(i + 1) % 3]).astype(np.uint32)
        x1 = (x1 + ks[(i + 2) % 3] + np.uint32(i + 1)).astype(np.uint32)
    return x0, x1


def _fixed_noise(B, N):
    fk0, fk1 = _threefry2x32(0, 0, np.zeros(1, np.uint32), np.ones(1, np.uint32))
    n = B * N
    x0, x1 = _threefry2x32(
        fk0[0], fk1[0], np.zeros(n, np.uint32), np.arange(n, dtype=np.uint32)
    )
    bits = x0 ^ x1
    f = ((bits >> np.uint32(9)) | np.uint32(0x3F800000)).view(np.float32)
    return np.maximum(np.float32(0.0), f - np.float32(1.0)).reshape(B, N)


@functools.lru_cache(maxsize=None)
def _shuffle_constants(B, N):
    noise = _fixed_noise(B, N)
    n_keep = int(N * (1 - _RATIO))
    ids_shuffle = np.argsort(noise, axis=1, kind="stable").astype(np.int32)
    ids_restore = np.argsort(ids_shuffle, axis=1, kind="stable").astype(np.int32)
    ids_keep = ids_shuffle[:, :n_keep]
    mask = np.ones((B, N), dtype=np.float32)
    mask[:, :n_keep] = 0.0
    mask = np.take_along_axis(mask, ids_restore, axis=1)
    flat_ids = (np.arange(B, dtype=np.int32)[:, None] * N + ids_keep).reshape(-1)
    return ids_keep, ids_restore, mask, flat_ids


_NBUF = 3


@functools.lru_cache(maxsize=None)
def _make_gather(R, D, ch):
    info = plsc.get_sparse_core_info()
    NC, NS = info.num_cores, info.num_subcores
    NW = NC * NS
    rpw = R // NW
    assert R % NW == 0 and rpw % ch == 0 and ch % 8 == 0 and ch <= 128
    nch = rpw // ch
    nbuf = min(_NBUF, nch)
    mesh = plsc.VectorSubcoreMesh(core_axis_name="c", subcore_axis_name="s")

    @functools.partial(
        pl.kernel,
        out_type=jax.ShapeDtypeStruct((R, D), jnp.float32),
        mesh=mesh,
        scratch_types=[
            pltpu.VMEM((nch, ch), jnp.int32),
        ]
        + [pltpu.VMEM((ch, D), jnp.float32)] * nbuf
        + [pltpu.SemaphoreType.DMA] * (2 * nbuf),
    )
    def gather(table_hbm, idx_hbm, out_hbm, idx_v, *bufs_sems):
        bufs = bufs_sems[:nbuf]
        gsem = bufs_sems[nbuf : 2 * nbuf]
        ssem = bufs_sems[2 * nbuf :]
        wid = lax.axis_index("s") * NC + lax.axis_index("c")
        base = wid * rpw
        pltpu.sync_copy(idx_hbm.at[wid], idx_v)
        gcp = [None] * nch
        scp = [None] * nch
        for c in range(nbuf):
            gcp[c] = pltpu.async_copy(
                table_hbm.at[idx_v.at[c]], bufs[c], gsem[c]
            )
        for c in range(nch):
            gcp[c].wait()
            scp[c] = pltpu.async_copy(
                bufs[c % nbuf], out_hbm.at[pl.ds(base + c * ch, ch)], ssem[c % nbuf]
            )
            if c + nbuf < nch:
                scp[c].wait()
                gcp[c + nbuf] = pltpu.async_copy(
                    table_hbm.at[idx_v.at[c + nbuf]], bufs[c % nbuf], gsem[c % nbuf]
                )
        for c in range(max(0, nch - nbuf), nch):
            scp[c].wait()

    return gather


def kernel(patches):
    B, N, D = patches.shape
    n_keep = int(N * (1 - _RATIO))
    ids_keep, ids_restore, mask, flat_ids = _shuffle_constants(B, N)
    R = B * n_keep
    info = plsc.get_sparse_core_info()
    NW = info.num_cores * info.num_subcores
    rpw = R // NW
    ch = max(
        c
        for c in range(8, min(rpw, 128) + 1, 8)
        if rpw % c == 0 and (_NBUF * c * D + rpw) * 4 <= 500_000
    )
    idx = jnp.asarray(flat_ids.reshape(NW, rpw // ch, ch))
    table = patches.reshape(B * N, D)
    out = _make_gather(R, D, ch)(table, idx)
    patches_keep = out.reshape(B, n_keep, D)
    return (
        patches_keep,
        jnp.asarray(mask),
        jnp.asarray(ids_keep),
        jnp.asarray(ids_restore),
    )
